# baseline (speedup 1.0000x reference)
import functools

import jax
import jax.numpy as jnp
import numpy as np
from jax.experimental import pallas as pl
from jax.experimental.pallas import tpu as pltpu

_N = 10000
_E = 320000
_B = 500
_D = 128
_NRBF = 64
_RCUT = 5.0
_MAXZ = 100

_LOG2 = float(np.log(2.0))
_BP = 512
_BN = 2000
_BE = 4000

_OFFS = np.linspace(0.0, _RCUT, _NRBF).astype(np.float32)
_WIDTH = float(_OFFS[1] - _OFFS[0])
_COEF = -0.5 / (_WIDTH * _WIDTH)


def _ssp(x):
    return jnp.maximum(x, 0.0) + jnp.log1p(jnp.exp(-jnp.abs(x))) - _LOG2


def _embed_body(z_ref, emb_ref, o_ref):
    zc = z_ref[:, 0][:, None]
    lane = jax.lax.broadcasted_iota(jnp.int32, (_BN, 128), 1)
    oh = (zc == lane).astype(jnp.float32)
    o_ref[...] = jnp.dot(oh, emb_ref[...], preferred_element_type=jnp.float32)


def _embed(z2, embp):
    return pl.pallas_call(
        _embed_body,
        grid=(_N // _BN,),
        in_specs=[
            pl.BlockSpec((_BN, 1), lambda i: (i, 0)),
            pl.BlockSpec((128, _D), lambda i: (0, 0)),
        ],
        out_specs=pl.BlockSpec((_BN, _D), lambda i: (i, 0)),
        out_shape=jax.ShapeDtypeStruct((_N, _D), jnp.float32),
    )(z2, embp)


def _wm_body(r2_ref, xd_ref, fw1_ref, fb1_ref, fw2_ref, fb2_ref, o_ref):
    r = jnp.sqrt(r2_ref[:, 0] + 1e-12)
    offs = (jax.lax.broadcasted_iota(jnp.int32, (_BE, _NRBF), 1)
            .astype(jnp.float32) * _WIDTH)
    e = jnp.exp(_COEF * (r[:, None] - offs) ** 2)
    fc = 0.5 * (jnp.cos((np.pi / _RCUT) * r) + 1.0) * (r < _RCUT).astype(jnp.float32)
    e = e * fc[:, None]
    a = _ssp(jnp.dot(e, fw1_ref[...], preferred_element_type=jnp.float32)
             + fb1_ref[...])
    w = jnp.dot(a, fw2_ref[...], preferred_element_type=jnp.float32) + fb2_ref[...]
    o_ref[...] = w * xd_ref[...]


def _wm(r2c, xd, fw1, fb1, fw2, fb2):
    return pl.pallas_call(
        _wm_body,
        grid=(_E // _BE,),
        in_specs=[
            pl.BlockSpec((_BE, 1), lambda i: (i, 0)),
            pl.BlockSpec((_BE, _D), lambda i: (i, 0)),
            pl.BlockSpec((_NRBF, _D), lambda i: (0, 0)),
            pl.BlockSpec((1, _D), lambda i: (0, 0)),
            pl.BlockSpec((_D, _D), lambda i: (0, 0)),
            pl.BlockSpec((1, _D), lambda i: (0, 0)),
        ],
        out_specs=pl.BlockSpec((_BE, _D), lambda i: (i, 0)),
        out_shape=jax.ShapeDtypeStruct((_E, _D), jnp.float32),
    )(r2c, xd, fw1, fb1, fw2, fb2)


def _upd_body(x_ref, mi_ref, uw1_ref, ub1_ref, uw2_ref, ub2_ref, o_ref):
    t = _ssp(jnp.dot(mi_ref[...], uw1_ref[...], preferred_element_type=jnp.float32)
             + ub1_ref[...])
    h = jnp.dot(t, uw2_ref[...], preferred_element_type=jnp.float32) + ub2_ref[...]
    o_ref[...] = x_ref[...] + h


def _upd(x, mi, uw1, ub1, uw2, ub2):
    return pl.pallas_call(
        _upd_body,
        grid=(_N // _BN,),
        in_specs=[
            pl.BlockSpec((_BN, _D), lambda i: (i, 0)),
            pl.BlockSpec((_BN, _D), lambda i: (i, 0)),
            pl.BlockSpec((_D, _D), lambda i: (0, 0)),
            pl.BlockSpec((1, _D), lambda i: (0, 0)),
            pl.BlockSpec((_D, _D), lambda i: (0, 0)),
            pl.BlockSpec((1, _D), lambda i: (0, 0)),
        ],
        out_specs=pl.BlockSpec((_BN, _D), lambda i: (i, 0)),
        out_shape=jax.ShapeDtypeStruct((_N, _D), jnp.float32),
    )(x, mi, uw1, ub1, uw2, ub2)


def _ro1_body(x_ref, bidx_ref, w1_ref, b1_ref, w2_ref, b2_ref,
              qraw_ref, psum_ref):
    i = pl.program_id(0)
    t = _ssp(jnp.dot(x_ref[...], w1_ref[...], preferred_element_type=jnp.float32)
             + b1_ref[...])
    u = jnp.dot(t, w2_ref[...], preferred_element_type=jnp.float32) + b2_ref[...]
    eps = u[:, 0]
    q = u[:, 1]
    qraw_ref[...] = q[:, None]
    lane = jax.lax.broadcasted_iota(jnp.int32, (_BN, 128), 1)
    vals = jnp.where(lane == 0, eps[:, None],
                     jnp.where(lane == 1, q[:, None],
                               jnp.where(lane == 2, 1.0, 0.0)))
    mol = jax.lax.broadcasted_iota(jnp.int32, (_BN, _BP), 1)
    oh = (bidx_ref[:, 0][:, None] == mol).astype(jnp.float32)

    @pl.when(i == 0)
    def _():
        psum_ref[...] = jnp.zeros_like(psum_ref)

    psum_ref[...] += jax.lax.dot_general(
        oh, vals, (((0,), (0,)), ((), ())),
        preferred_element_type=jnp.float32)


def _ro1(x, bidx2, w1, b1, w2, b2):
    return pl.pallas_call(
        _ro1_body,
        grid=(_N // _BN,),
        in_specs=[
            pl.BlockSpec((_BN, _D), lambda i: (i, 0)),
            pl.BlockSpec((_BN, 1), lambda i: (i, 0)),
            pl.BlockSpec((_D, _D), lambda i: (0, 0)),
            pl.BlockSpec((1, _D), lambda i: (0, 0)),
            pl.BlockSpec((_D, 128), lambda i: (0, 0)),
            pl.BlockSpec((1, 128), lambda i: (0, 0)),
        ],
        out_specs=[
            pl.BlockSpec((_BN, 1), lambda i: (i, 0)),
            pl.BlockSpec((_BP, 128), lambda i: (0, 0)),
        ],
        out_shape=[
            jax.ShapeDtypeStruct((_N, 1), jnp.float32),
            jax.ShapeDtypeStruct((_BP, 128), jnp.float32),
        ],
    )(x, bidx2, w1, b1, w2, b2)


def _ro2_body(qraw_ref, bidx_ref, posp_ref, psum_ref,
              qi_ref, dip_ref, acc_ref):
    i = pl.program_id(0)
    nsteps = pl.num_programs(0)
    molq = psum_ref[:, 1]
    nat = psum_ref[:, 2]
    meanq = molq / jnp.maximum(nat, 1.0)
    lane = jax.lax.broadcasted_iota(jnp.int32, (_BP, 8), 1)
    meanq_mat = jnp.where(lane == 0, meanq[:, None], 0.0)
    mol = jax.lax.broadcasted_iota(jnp.int32, (_BN, _BP), 1)
    oh = (bidx_ref[:, 0][:, None] == mol).astype(jnp.float32)
    mq = jnp.dot(oh, meanq_mat, preferred_element_type=jnp.float32)[:, 0]
    q_i = qraw_ref[:, 0] - mq
    qi_ref[...] = q_i[:, None]
    vals = q_i[:, None] * posp_ref[...]

    @pl.when(i == 0)
    def _():
        acc_ref[...] = jnp.zeros_like(acc_ref)

    acc_ref[...] += jax.lax.dot_general(
        oh, vals, (((0,), (0,)), ((), ())),
        preferred_element_type=jnp.float32)

    @pl.when(i == nsteps - 1)
    def _():
        mu = acc_ref[...]
        dip_ref[...] = jnp.sqrt(jnp.sum(mu * mu, axis=1) + 1e-12)[:, None]


def _ro2(qraw, bidx2, posp, psum):
    return pl.pallas_call(
        _ro2_body,
        grid=(_N // _BN,),
        in_specs=[
            pl.BlockSpec((_BN, 1), lambda i: (i, 0)),
            pl.BlockSpec((_BN, 1), lambda i: (i, 0)),
            pl.BlockSpec((_BN, 8), lambda i: (i, 0)),
            pl.BlockSpec((_BP, 128), lambda i: (0, 0)),
        ],
        out_specs=[
            pl.BlockSpec((_BN, 1), lambda i: (i, 0)),
            pl.BlockSpec((_BP, 1), lambda i: (0, 0)),
        ],
        out_shape=[
            jax.ShapeDtypeStruct((_N, 1), jnp.float32),
            jax.ShapeDtypeStruct((_BP, 1), jnp.float32),
        ],
        scratch_shapes=[pltpu.VMEM((_BP, 8), jnp.float32)],
    )(qraw, bidx2, posp, psum)


def kernel(z, pos, edge_index, batch_idx, params):
    src = edge_index[0]
    dst = edge_index[1]

    d = pos[dst] - pos[src]
    r2 = jnp.sum(d * d, axis=-1)

    embp = jnp.zeros((128, _D), jnp.float32).at[: _MAXZ + 1].set(params["emb"])
    x = _embed(z.astype(jnp.int32).reshape(_N, 1), embp)

    r2c = r2.reshape(_E, 1)
    for blk in params["blocks"]:
        xd = x[dst]
        m = _wm(r2c, xd,
                blk["fw1"], blk["fb1"].reshape(1, _D),
                blk["fw2"], blk["fb2"].reshape(1, _D))
        mi = jax.ops.segment_sum(m, src, num_segments=_N)
        x = _upd(x, mi,
                 blk["uw1"], blk["ub1"].reshape(1, _D),
                 blk["uw2"], blk["ub2"].reshape(1, _D))

    w1 = jnp.concatenate([params["ew1"], params["cw1"]], axis=1)
    b1 = jnp.concatenate([params["eb1"], params["cb1"]], axis=0).reshape(1, 128)
    w2 = jnp.zeros((_D, 128), jnp.float32)
    w2 = w2.at[: _D // 2, 0].set(params["ew2"][:, 0])
    w2 = w2.at[_D // 2 :, 1].set(params["cw2"][:, 0])
    b2 = jnp.zeros((1, 128), jnp.float32)
    b2 = b2.at[0, 0].set(params["eb2"][0])
    b2 = b2.at[0, 1].set(params["cb2"][0])

    bidx2 = batch_idx.astype(jnp.int32).reshape(_N, 1)
    qraw, psum = _ro1(x, bidx2, w1, b1, w2, b2)

    posp = jnp.zeros((_N, 8), jnp.float32).at[:, :3].set(pos)
    qi, dip = _ro2(qraw, bidx2, posp, psum)

    energy = psum[: _B, 0]
    dipole = dip[: _B, 0]
    q_i = qi[:, 0]
    return energy, dipole, q_i


# speedup vs baseline: 1.7546x; 1.7546x over previous
import functools

import jax
import jax.numpy as jnp
import numpy as np
from jax import lax
from jax.experimental import pallas as pl
from jax.experimental.pallas import tpu as pltpu
from jax.experimental.pallas import tpu_sc as plsc

_N = 10000
_E = 320000
_B = 500
_D = 128
_NRBF = 64
_RCUT = 5.0
_MAXZ = 100

_LOG2 = float(np.log(2.0))
_BP = 512
_BN = 2000
_BE = 4096

_OFFS = np.linspace(0.0, _RCUT, _NRBF).astype(np.float32)
_WIDTH = float(_OFFS[1] - _OFFS[0])
_COEF = -0.5 / (_WIDTH * _WIDTH)


def _ssp(x):
    return jnp.maximum(x, 0.0) + jnp.log1p(jnp.exp(-jnp.abs(x))) - _LOG2


_NC = 2
_NS = 16
_NW = _NC * _NS
_CH = 128
_CPT = 80
_NCHUNK = _NW * _CPT
_EPAD = _NCHUNK * _CH
_EPT = _E // _NW
_NPAD = 10240
_RPS = _NPAD // _NS

_sc_mesh = plsc.VectorSubcoreMesh(core_axis_name="c", subcore_axis_name="s")
_sc_params = pltpu.CompilerParams(needs_layout_passes=False)


def _wid():
    return lax.axis_index("s") * _NC + lax.axis_index("c")


def _r2_body(px_hbm, py_hbm, pz_hbm, src_hbm, dst_hbm, out_hbm,
             px, py, pz, srcv, dstv, outv):
    w = _wid()
    base = w * _EPT
    pltpu.sync_copy(px_hbm, px)
    pltpu.sync_copy(py_hbm, py)
    pltpu.sync_copy(pz_hbm, pz)
    pltpu.sync_copy(src_hbm.at[pl.ds(base, _EPT)], srcv)
    pltpu.sync_copy(dst_hbm.at[pl.ds(base, _EPT)], dstv)

    def body(j, carry):
        sl = pl.ds(j * 16, 16)
        si = srcv[sl]
        di = dstv[sl]
        dx = plsc.load_gather(px, [di]) - plsc.load_gather(px, [si])
        dy = plsc.load_gather(py, [di]) - plsc.load_gather(py, [si])
        dz = plsc.load_gather(pz, [di]) - plsc.load_gather(pz, [si])
        outv[sl] = dx * dx + dy * dy + dz * dz
        return carry

    lax.fori_loop(0, _EPT // 16, body, 0)
    pltpu.sync_copy(outv, out_hbm.at[pl.ds(base, _EPT)])


def _sc_r2(px, py, pz, src, dst):
    f = functools.partial(
        pl.kernel,
        out_type=jax.ShapeDtypeStruct((_E,), jnp.float32),
        mesh=_sc_mesh,
        compiler_params=_sc_params,
        scratch_types=[
            pltpu.VMEM((_N,), jnp.float32),
            pltpu.VMEM((_N,), jnp.float32),
            pltpu.VMEM((_N,), jnp.float32),
            pltpu.VMEM((_EPT,), jnp.int32),
            pltpu.VMEM((_EPT,), jnp.int32),
            pltpu.VMEM((_EPT,), jnp.float32),
        ],
    )(_r2_body)
    return f(px, py, pz, src, dst)


def _gather_body(x_hbm, dst2_hbm, out_hbm, idx_v, rows_v, sem):
    w = _wid()
    pltpu.sync_copy(dst2_hbm.at[pl.ds(w * _CPT, _CPT)], idx_v)

    def body(g, carry):
        c = w * _CPT + g
        pltpu.async_copy(x_hbm.at[idx_v.at[g]], rows_v, sem).wait()
        pltpu.sync_copy(rows_v, out_hbm.at[pl.ds(c * _CH, _CH)])
        return carry

    lax.fori_loop(0, _CPT, body, 0)


def _sc_gather(x, dst2):
    f = functools.partial(
        pl.kernel,
        out_type=jax.ShapeDtypeStruct((_EPAD, _D), jnp.float32),
        mesh=_sc_mesh,
        compiler_params=_sc_params,
        scratch_types=[
            pltpu.VMEM((_CPT, _CH), jnp.int32),
            pltpu.VMEM((_CH, _D), jnp.float32),
            pltpu.SemaphoreType.DMA,
        ],
    )(_gather_body)
    return f(x, dst2)


def _scatter_body(m_hbm, src2_hbm, zeros_hbm, p0_hbm, p1_hbm,
                  acc, idx_v, mbuf):
    c = lax.axis_index("c")
    s = lax.axis_index("s")
    w = s * _NC + c
    rsl = pl.ds(s * _RPS, _RPS)
    pltpu.sync_copy(zeros_hbm, acc.at[rsl])
    plsc.subcore_barrier()

    pltpu.sync_copy(src2_hbm.at[pl.ds(w * _CPT, _CPT)], idx_v)

    def body(g, carry):
        cid = w * _CPT + g
        pltpu.sync_copy(m_hbm.at[pl.ds(cid * _CH, _CH)], mbuf)
        pltpu.sync_copy(mbuf, acc.at[idx_v.at[g]], add=True)
        return carry

    lax.fori_loop(0, _CPT, body, 0)

    plsc.subcore_barrier()

    @pl.when(c == 0)
    def _():
        pltpu.sync_copy(acc.at[rsl], p0_hbm.at[rsl])

    @pl.when(c == 1)
    def _():
        pltpu.sync_copy(acc.at[rsl], p1_hbm.at[rsl])


def _sc_scatter(m, src2, zeros):
    f = functools.partial(
        pl.kernel,
        out_type=(
            jax.ShapeDtypeStruct((_NPAD, _D), jnp.float32),
            jax.ShapeDtypeStruct((_NPAD, _D), jnp.float32),
        ),
        mesh=_sc_mesh,
        compiler_params=_sc_params,
        scratch_types=[
            pltpu.VMEM_SHARED((_NPAD, _D), jnp.float32),
            pltpu.VMEM((_CPT, _CH), jnp.int32),
            pltpu.VMEM((_CH, _D), jnp.float32),
        ],
    )(_scatter_body)
    return f(m, src2, zeros)


def _embed_body(z_ref, emb_ref, o_ref):
    zc = z_ref[:, 0][:, None]
    lane = jax.lax.broadcasted_iota(jnp.int32, (_BN, 128), 1)
    oh = (zc == lane).astype(jnp.float32)
    o_ref[...] = jnp.dot(oh, emb_ref[...], preferred_element_type=jnp.float32)


def _embed(z2, embp):
    return pl.pallas_call(
        _embed_body,
        grid=(_N // _BN,),
        in_specs=[
            pl.BlockSpec((_BN, 1), lambda i: (i, 0)),
            pl.BlockSpec((128, _D), lambda i: (0, 0)),
        ],
        out_specs=pl.BlockSpec((_BN, _D), lambda i: (i, 0)),
        out_shape=jax.ShapeDtypeStruct((_N, _D), jnp.float32),
    )(z2, embp)


def _wm_body(r2_ref, xd_ref, fw1_ref, fb1_ref, fw2_ref, fb2_ref, o_ref):
    r = jnp.sqrt(r2_ref[:, 0] + 1e-12)
    offs = (jax.lax.broadcasted_iota(jnp.int32, (_BE, _NRBF), 1)
            .astype(jnp.float32) * _WIDTH)
    e = jnp.exp(_COEF * (r[:, None] - offs) ** 2)
    fc = 0.5 * (jnp.cos((np.pi / _RCUT) * r) + 1.0) * (r < _RCUT).astype(jnp.float32)
    e = e * fc[:, None]
    a = _ssp(jnp.dot(e, fw1_ref[...], preferred_element_type=jnp.float32)
             + fb1_ref[...])
    w = jnp.dot(a, fw2_ref[...], preferred_element_type=jnp.float32) + fb2_ref[...]
    o_ref[...] = w * xd_ref[...]


def _wm(r2c, xd, fw1, fb1, fw2, fb2):
    return pl.pallas_call(
        _wm_body,
        grid=(_EPAD // _BE,),
        in_specs=[
            pl.BlockSpec((_BE, 1), lambda i: (i, 0)),
            pl.BlockSpec((_BE, _D), lambda i: (i, 0)),
            pl.BlockSpec((_NRBF, _D), lambda i: (0, 0)),
            pl.BlockSpec((1, _D), lambda i: (0, 0)),
            pl.BlockSpec((_D, _D), lambda i: (0, 0)),
            pl.BlockSpec((1, _D), lambda i: (0, 0)),
        ],
        out_specs=pl.BlockSpec((_BE, _D), lambda i: (i, 0)),
        out_shape=jax.ShapeDtypeStruct((_EPAD, _D), jnp.float32),
    )(r2c, xd, fw1, fb1, fw2, fb2)


def _upd_body(x_ref, p0_ref, p1_ref, uw1_ref, ub1_ref, uw2_ref, ub2_ref, o_ref):
    mi = p0_ref[...] + p1_ref[...]
    t = _ssp(jnp.dot(mi, uw1_ref[...], preferred_element_type=jnp.float32)
             + ub1_ref[...])
    h = jnp.dot(t, uw2_ref[...], preferred_element_type=jnp.float32) + ub2_ref[...]
    o_ref[...] = x_ref[...] + h


def _upd(x, p0, p1, uw1, ub1, uw2, ub2):
    return pl.pallas_call(
        _upd_body,
        grid=(_N // _BN,),
        in_specs=[
            pl.BlockSpec((_BN, _D), lambda i: (i, 0)),
            pl.BlockSpec((_BN, _D), lambda i: (i, 0)),
            pl.BlockSpec((_BN, _D), lambda i: (i, 0)),
            pl.BlockSpec((_D, _D), lambda i: (0, 0)),
            pl.BlockSpec((1, _D), lambda i: (0, 0)),
            pl.BlockSpec((_D, _D), lambda i: (0, 0)),
            pl.BlockSpec((1, _D), lambda i: (0, 0)),
        ],
        out_specs=pl.BlockSpec((_BN, _D), lambda i: (i, 0)),
        out_shape=jax.ShapeDtypeStruct((_N, _D), jnp.float32),
    )(x, p0, p1, uw1, ub1, uw2, ub2)


def _ro1_body(x_ref, bidx_ref, w1_ref, b1_ref, w2_ref, b2_ref,
              qraw_ref, psum_ref):
    i = pl.program_id(0)
    t = _ssp(jnp.dot(x_ref[...], w1_ref[...], preferred_element_type=jnp.float32)
             + b1_ref[...])
    u = jnp.dot(t, w2_ref[...], preferred_element_type=jnp.float32) + b2_ref[...]
    eps = u[:, 0]
    q = u[:, 1]
    qraw_ref[...] = q[:, None]
    lane = jax.lax.broadcasted_iota(jnp.int32, (_BN, 128), 1)
    vals = jnp.where(lane == 0, eps[:, None],
                     jnp.where(lane == 1, q[:, None],
                               jnp.where(lane == 2, 1.0, 0.0)))
    mol = jax.lax.broadcasted_iota(jnp.int32, (_BN, _BP), 1)
    oh = (bidx_ref[:, 0][:, None] == mol).astype(jnp.float32)

    @pl.when(i == 0)
    def _():
        psum_ref[...] = jnp.zeros_like(psum_ref)

    psum_ref[...] += jax.lax.dot_general(
        oh, vals, (((0,), (0,)), ((), ())),
        preferred_element_type=jnp.float32)


def _ro1(x, bidx2, w1, b1, w2, b2):
    return pl.pallas_call(
        _ro1_body,
        grid=(_N // _BN,),
        in_specs=[
            pl.BlockSpec((_BN, _D), lambda i: (i, 0)),
            pl.BlockSpec((_BN, 1), lambda i: (i, 0)),
            pl.BlockSpec((_D, _D), lambda i: (0, 0)),
            pl.BlockSpec((1, _D), lambda i: (0, 0)),
            pl.BlockSpec((_D, 128), lambda i: (0, 0)),
            pl.BlockSpec((1, 128), lambda i: (0, 0)),
        ],
        out_specs=[
            pl.BlockSpec((_BN, 1), lambda i: (i, 0)),
            pl.BlockSpec((_BP, 128), lambda i: (0, 0)),
        ],
        out_shape=[
            jax.ShapeDtypeStruct((_N, 1), jnp.float32),
            jax.ShapeDtypeStruct((_BP, 128), jnp.float32),
        ],
    )(x, bidx2, w1, b1, w2, b2)


def _ro2_body(qraw_ref, bidx_ref, posp_ref, psum_ref,
              qi_ref, dip_ref, acc_ref):
    i = pl.program_id(0)
    nsteps = pl.num_programs(0)
    molq = psum_ref[:, 1]
    nat = psum_ref[:, 2]
    meanq = molq / jnp.maximum(nat, 1.0)
    lane = jax.lax.broadcasted_iota(jnp.int32, (_BP, 8), 1)
    meanq_mat = jnp.where(lane == 0, meanq[:, None], 0.0)
    mol = jax.lax.broadcasted_iota(jnp.int32, (_BN, _BP), 1)
    oh = (bidx_ref[:, 0][:, None] == mol).astype(jnp.float32)
    mq = jnp.dot(oh, meanq_mat, preferred_element_type=jnp.float32)[:, 0]
    q_i = qraw_ref[:, 0] - mq
    qi_ref[...] = q_i[:, None]
    vals = q_i[:, None] * posp_ref[...]

    @pl.when(i == 0)
    def _():
        acc_ref[...] = jnp.zeros_like(acc_ref)

    acc_ref[...] += jax.lax.dot_general(
        oh, vals, (((0,), (0,)), ((), ())),
        preferred_element_type=jnp.float32)

    @pl.when(i == nsteps - 1)
    def _():
        mu = acc_ref[...]
        dip_ref[...] = jnp.sqrt(jnp.sum(mu * mu, axis=1) + 1e-12)[:, None]


def _ro2(qraw, bidx2, posp, psum):
    return pl.pallas_call(
        _ro2_body,
        grid=(_N // _BN,),
        in_specs=[
            pl.BlockSpec((_BN, 1), lambda i: (i, 0)),
            pl.BlockSpec((_BN, 1), lambda i: (i, 0)),
            pl.BlockSpec((_BN, 8), lambda i: (i, 0)),
            pl.BlockSpec((_BP, 128), lambda i: (0, 0)),
        ],
        out_specs=[
            pl.BlockSpec((_BN, 1), lambda i: (i, 0)),
            pl.BlockSpec((_BP, 1), lambda i: (0, 0)),
        ],
        out_shape=[
            jax.ShapeDtypeStruct((_N, 1), jnp.float32),
            jax.ShapeDtypeStruct((_BP, 1), jnp.float32),
        ],
        scratch_shapes=[pltpu.VMEM((_BP, 8), jnp.float32)],
    )(qraw, bidx2, posp, psum)


def kernel(z, pos, edge_index, batch_idx, params):
    src = edge_index[0].astype(jnp.int32)
    dst = edge_index[1].astype(jnp.int32)
    npad = _EPAD - _E
    src2 = jnp.concatenate(
        [src, jnp.full((npad,), _NPAD - 1, jnp.int32)]).reshape(_NCHUNK, _CH)
    dst2 = jnp.concatenate(
        [dst, jnp.zeros((npad,), jnp.int32)]).reshape(_NCHUNK, _CH)
    px = jnp.asarray(pos[:, 0], jnp.float32)
    py = jnp.asarray(pos[:, 1], jnp.float32)
    pz = jnp.asarray(pos[:, 2], jnp.float32)
    zeros = jnp.zeros((_RPS, _D), jnp.float32)

    r2 = jnp.concatenate(
        [_sc_r2(px, py, pz, src, dst), jnp.zeros((_EPAD - _E,), jnp.float32)])

    embp = jnp.zeros((128, _D), jnp.float32).at[: _MAXZ + 1].set(params["emb"])
    x = _embed(z.astype(jnp.int32).reshape(_N, 1), embp)

    r2c = r2.reshape(_EPAD, 1)
    for blk in params["blocks"]:
        xd = _sc_gather(x, dst2)
        m = _wm(r2c, xd,
                blk["fw1"], blk["fb1"].reshape(1, _D),
                blk["fw2"], blk["fb2"].reshape(1, _D))
        p0, p1 = _sc_scatter(m, src2, zeros)
        x = _upd(x, p0, p1,
                 blk["uw1"], blk["ub1"].reshape(1, _D),
                 blk["uw2"], blk["ub2"].reshape(1, _D))

    w1 = jnp.concatenate([params["ew1"], params["cw1"]], axis=1)
    b1 = jnp.concatenate([params["eb1"], params["cb1"]], axis=0).reshape(1, 128)
    w2 = jnp.zeros((_D, 128), jnp.float32)
    w2 = w2.at[: _D // 2, 0].set(params["ew2"][:, 0])
    w2 = w2.at[_D // 2 :, 1].set(params["cw2"][:, 0])
    b2 = jnp.zeros((1, 128), jnp.float32)
    b2 = b2.at[0, 0].set(params["eb2"][0])
    b2 = b2.at[0, 1].set(params["cb2"][0])

    bidx2 = batch_idx.astype(jnp.int32).reshape(_N, 1)
    qraw, psum = _ro1(x, bidx2, w1, b1, w2, b2)

    posp = jnp.zeros((_N, 8), jnp.float32).at[:, :3].set(pos)
    qi, dip = _ro2(qraw, bidx2, posp, psum)

    energy = psum[: _B, 0]
    dipole = dip[: _B, 0]
    q_i = qi[:, 0]
    return energy, dipole, q_i


# speedup vs baseline: 1.9701x; 1.1228x over previous
import functools

import jax
import jax.numpy as jnp
import numpy as np
from jax import lax
from jax.experimental import pallas as pl
from jax.experimental.pallas import tpu as pltpu
from jax.experimental.pallas import tpu_sc as plsc

_N = 10000
_E = 320000
_B = 500
_D = 128
_NRBF = 64
_RCUT = 5.0
_MAXZ = 100

_LOG2 = float(np.log(2.0))
_BP = 512
_BN = 2000
_BE = 4096

_OFFS = np.linspace(0.0, _RCUT, _NRBF).astype(np.float32)
_WIDTH = float(_OFFS[1] - _OFFS[0])
_COEF = -0.5 / (_WIDTH * _WIDTH)


def _ssp(x):
    return jnp.maximum(x, 0.0) + jnp.log1p(jnp.exp(-jnp.abs(x))) - _LOG2


_NC = 2
_NS = 16
_NW = _NC * _NS
_CH = 128
_CPT = 80
_NCHUNK = _NW * _CPT
_EPAD = _NCHUNK * _CH
_EPT = _E // _NW
_NPAD = 10240
_RPS = _NPAD // _NS

_sc_mesh = plsc.VectorSubcoreMesh(core_axis_name="c", subcore_axis_name="s")
_sc_params = pltpu.CompilerParams(needs_layout_passes=False)


def _wid():
    return lax.axis_index("s") * _NC + lax.axis_index("c")


def _r2_body(px_hbm, py_hbm, pz_hbm, src_hbm, dst_hbm, out_hbm,
             px, py, pz, srcv, dstv, outv):
    w = _wid()
    base = w * _EPT
    pltpu.sync_copy(px_hbm, px)
    pltpu.sync_copy(py_hbm, py)
    pltpu.sync_copy(pz_hbm, pz)
    pltpu.sync_copy(src_hbm.at[pl.ds(base, _EPT)], srcv)
    pltpu.sync_copy(dst_hbm.at[pl.ds(base, _EPT)], dstv)

    def body(j, carry):
        sl = pl.ds(j * 16, 16)
        si = srcv[sl]
        di = dstv[sl]
        dx = plsc.load_gather(px, [di]) - plsc.load_gather(px, [si])
        dy = plsc.load_gather(py, [di]) - plsc.load_gather(py, [si])
        dz = plsc.load_gather(pz, [di]) - plsc.load_gather(pz, [si])
        outv[sl] = dx * dx + dy * dy + dz * dz
        return carry

    lax.fori_loop(0, _EPT // 16, body, 0)
    pltpu.sync_copy(outv, out_hbm.at[pl.ds(base, _EPT)])


def _sc_r2(px, py, pz, src, dst):
    f = functools.partial(
        pl.kernel,
        out_type=jax.ShapeDtypeStruct((_E,), jnp.float32),
        mesh=_sc_mesh,
        compiler_params=_sc_params,
        scratch_types=[
            pltpu.VMEM((_N,), jnp.float32),
            pltpu.VMEM((_N,), jnp.float32),
            pltpu.VMEM((_N,), jnp.float32),
            pltpu.VMEM((_EPT,), jnp.int32),
            pltpu.VMEM((_EPT,), jnp.int32),
            pltpu.VMEM((_EPT,), jnp.float32),
        ],
    )(_r2_body)
    return f(px, py, pz, src, dst)


_NBUF = 4


def _gather_body(x_hbm, dst2_hbm, out_hbm, idx_v, *bufs_sems):
    rbs = bufs_sems[:_NBUF]
    sgs = bufs_sems[_NBUF:2 * _NBUF]
    sos = bufs_sems[2 * _NBUF:3 * _NBUF]
    w = _wid()
    base = w * _CPT
    pltpu.sync_copy(dst2_hbm.at[pl.ds(base, _CPT)], idx_v)

    def out_sl(g):
        return out_hbm.at[pl.ds((base + g) * _CH, _CH)]

    for b in range(_NBUF):
        pltpu.async_copy(x_hbm.at[idx_v.at[b]], rbs[b], sgs[b])

    def body(k, carry):
        for b in range(_NBUF):
            g = k * _NBUF + b
            pltpu.make_async_copy(x_hbm.at[idx_v.at[g]], rbs[b], sgs[b]).wait()
            pltpu.async_copy(rbs[b], out_sl(g), sos[b])

            @pl.when(g + _NBUF < _CPT)
            def _():
                pltpu.make_async_copy(rbs[b], out_sl(g), sos[b]).wait()
                pltpu.async_copy(
                    x_hbm.at[idx_v.at[g + _NBUF]], rbs[b], sgs[b])
        return carry

    lax.fori_loop(0, _CPT // _NBUF, body, 0)
    for b in range(_NBUF):
        pltpu.make_async_copy(rbs[b], out_sl(_CPT - _NBUF + b), sos[b]).wait()


def _sc_gather(x, dst2):
    f = functools.partial(
        pl.kernel,
        out_type=jax.ShapeDtypeStruct((_EPAD, _D), jnp.float32),
        mesh=_sc_mesh,
        compiler_params=_sc_params,
        scratch_types=[
            pltpu.VMEM((_CPT, _CH), jnp.int32),
        ] + [pltpu.VMEM((_CH, _D), jnp.float32)] * _NBUF
          + [pltpu.SemaphoreType.DMA] * (2 * _NBUF),
    )(_gather_body)
    return f(x, dst2)


_NBUF_S = 2


def _scatter_body(m_hbm, src2_hbm, zeros_hbm, p0_hbm, p1_hbm,
                  acc, idx_v, *bufs_sems):
    mbs = bufs_sems[:_NBUF_S]
    sis = bufs_sems[_NBUF_S:2 * _NBUF_S]
    c = lax.axis_index("c")
    s = lax.axis_index("s")
    w = s * _NC + c
    base = w * _CPT
    rsl = pl.ds(s * _RPS, _RPS)
    pltpu.sync_copy(zeros_hbm, acc.at[rsl])
    plsc.subcore_barrier()

    pltpu.sync_copy(src2_hbm.at[pl.ds(base, _CPT)], idx_v)

    def m_sl(g):
        return m_hbm.at[pl.ds((base + g) * _CH, _CH)]

    for b in range(_NBUF_S):
        pltpu.async_copy(m_sl(b), mbs[b], sis[b])

    def body(k, carry):
        for b in range(_NBUF_S):
            g = k * _NBUF_S + b
            pltpu.make_async_copy(m_sl(g), mbs[b], sis[b]).wait()
            pltpu.sync_copy(mbs[b], acc.at[idx_v.at[g]], add=True)

            @pl.when(g + _NBUF_S < _CPT)
            def _():
                pltpu.async_copy(m_sl(g + _NBUF_S), mbs[b], sis[b])
        return carry

    lax.fori_loop(0, _CPT // _NBUF_S, body, 0)

    plsc.subcore_barrier()

    @pl.when(c == 0)
    def _():
        pltpu.sync_copy(acc.at[rsl], p0_hbm.at[rsl])

    @pl.when(c == 1)
    def _():
        pltpu.sync_copy(acc.at[rsl], p1_hbm.at[rsl])


def _sc_scatter(m, src2, zeros):
    f = functools.partial(
        pl.kernel,
        out_type=(
            jax.ShapeDtypeStruct((_NPAD, _D), jnp.float32),
            jax.ShapeDtypeStruct((_NPAD, _D), jnp.float32),
        ),
        mesh=_sc_mesh,
        compiler_params=_sc_params,
        scratch_types=[
            pltpu.VMEM_SHARED((_NPAD, _D), jnp.float32),
            pltpu.VMEM((_CPT, _CH), jnp.int32),
        ] + [pltpu.VMEM((_CH, _D), jnp.float32)] * _NBUF_S
          + [pltpu.SemaphoreType.DMA] * _NBUF_S,
    )(_scatter_body)
    return f(m, src2, zeros)


def _embed_body(z_ref, emb_ref, o_ref):
    zc = z_ref[:, 0][:, None]
    lane = jax.lax.broadcasted_iota(jnp.int32, (_BN, 128), 1)
    oh = (zc == lane).astype(jnp.float32)
    o_ref[...] = jnp.dot(oh, emb_ref[...], preferred_element_type=jnp.float32)


def _embed(z2, embp):
    return pl.pallas_call(
        _embed_body,
        grid=(_N // _BN,),
        in_specs=[
            pl.BlockSpec((_BN, 1), lambda i: (i, 0)),
            pl.BlockSpec((128, _D), lambda i: (0, 0)),
        ],
        out_specs=pl.BlockSpec((_BN, _D), lambda i: (i, 0)),
        out_shape=jax.ShapeDtypeStruct((_N, _D), jnp.float32),
    )(z2, embp)


def _wm_body(r2_ref, xd_ref, fw1_ref, fb1_ref, fw2_ref, fb2_ref, o_ref):
    r = jnp.sqrt(r2_ref[:, 0] + 1e-12)
    offs = (jax.lax.broadcasted_iota(jnp.int32, (_BE, _NRBF), 1)
            .astype(jnp.float32) * _WIDTH)
    e = jnp.exp(_COEF * (r[:, None] - offs) ** 2)
    fc = 0.5 * (jnp.cos((np.pi / _RCUT) * r) + 1.0) * (r < _RCUT).astype(jnp.float32)
    e = e * fc[:, None]
    a = _ssp(jnp.dot(e, fw1_ref[...], preferred_element_type=jnp.float32)
             + fb1_ref[...])
    w = jnp.dot(a, fw2_ref[...], preferred_element_type=jnp.float32) + fb2_ref[...]
    o_ref[...] = w * xd_ref[...]


def _wm(r2c, xd, fw1, fb1, fw2, fb2):
    return pl.pallas_call(
        _wm_body,
        grid=(_EPAD // _BE,),
        in_specs=[
            pl.BlockSpec((_BE, 1), lambda i: (i, 0)),
            pl.BlockSpec((_BE, _D), lambda i: (i, 0)),
            pl.BlockSpec((_NRBF, _D), lambda i: (0, 0)),
            pl.BlockSpec((1, _D), lambda i: (0, 0)),
            pl.BlockSpec((_D, _D), lambda i: (0, 0)),
            pl.BlockSpec((1, _D), lambda i: (0, 0)),
        ],
        out_specs=pl.BlockSpec((_BE, _D), lambda i: (i, 0)),
        out_shape=jax.ShapeDtypeStruct((_EPAD, _D), jnp.float32),
    )(r2c, xd, fw1, fb1, fw2, fb2)


def _upd_body(x_ref, p0_ref, p1_ref, uw1_ref, ub1_ref, uw2_ref, ub2_ref, o_ref):
    mi = p0_ref[...] + p1_ref[...]
    t = _ssp(jnp.dot(mi, uw1_ref[...], preferred_element_type=jnp.float32)
             + ub1_ref[...])
    h = jnp.dot(t, uw2_ref[...], preferred_element_type=jnp.float32) + ub2_ref[...]
    o_ref[...] = x_ref[...] + h


def _upd(x, p0, p1, uw1, ub1, uw2, ub2):
    return pl.pallas_call(
        _upd_body,
        grid=(_N // _BN,),
        in_specs=[
            pl.BlockSpec((_BN, _D), lambda i: (i, 0)),
            pl.BlockSpec((_BN, _D), lambda i: (i, 0)),
            pl.BlockSpec((_BN, _D), lambda i: (i, 0)),
            pl.BlockSpec((_D, _D), lambda i: (0, 0)),
            pl.BlockSpec((1, _D), lambda i: (0, 0)),
            pl.BlockSpec((_D, _D), lambda i: (0, 0)),
            pl.BlockSpec((1, _D), lambda i: (0, 0)),
        ],
        out_specs=pl.BlockSpec((_BN, _D), lambda i: (i, 0)),
        out_shape=jax.ShapeDtypeStruct((_N, _D), jnp.float32),
    )(x, p0, p1, uw1, ub1, uw2, ub2)


def _ro1_body(x_ref, bidx_ref, w1_ref, b1_ref, w2_ref, b2_ref,
              qraw_ref, psum_ref):
    i = pl.program_id(0)
    t = _ssp(jnp.dot(x_ref[...], w1_ref[...], preferred_element_type=jnp.float32)
             + b1_ref[...])
    u = jnp.dot(t, w2_ref[...], preferred_element_type=jnp.float32) + b2_ref[...]
    eps = u[:, 0]
    q = u[:, 1]
    qraw_ref[...] = q[:, None]
    lane = jax.lax.broadcasted_iota(jnp.int32, (_BN, 128), 1)
    vals = jnp.where(lane == 0, eps[:, None],
                     jnp.where(lane == 1, q[:, None],
                               jnp.where(lane == 2, 1.0, 0.0)))
    mol = jax.lax.broadcasted_iota(jnp.int32, (_BN, _BP), 1)
    oh = (bidx_ref[:, 0][:, None] == mol).astype(jnp.float32)

    @pl.when(i == 0)
    def _():
        psum_ref[...] = jnp.zeros_like(psum_ref)

    psum_ref[...] += jax.lax.dot_general(
        oh, vals, (((0,), (0,)), ((), ())),
        preferred_element_type=jnp.float32)


def _ro1(x, bidx2, w1, b1, w2, b2):
    return pl.pallas_call(
        _ro1_body,
        grid=(_N // _BN,),
        in_specs=[
            pl.BlockSpec((_BN, _D), lambda i: (i, 0)),
            pl.BlockSpec((_BN, 1), lambda i: (i, 0)),
            pl.BlockSpec((_D, _D), lambda i: (0, 0)),
            pl.BlockSpec((1, _D), lambda i: (0, 0)),
            pl.BlockSpec((_D, 128), lambda i: (0, 0)),
            pl.BlockSpec((1, 128), lambda i: (0, 0)),
        ],
        out_specs=[
            pl.BlockSpec((_BN, 1), lambda i: (i, 0)),
            pl.BlockSpec((_BP, 128), lambda i: (0, 0)),
        ],
        out_shape=[
            jax.ShapeDtypeStruct((_N, 1), jnp.float32),
            jax.ShapeDtypeStruct((_BP, 128), jnp.float32),
        ],
    )(x, bidx2, w1, b1, w2, b2)


def _ro2_body(qraw_ref, bidx_ref, posp_ref, psum_ref,
              qi_ref, dip_ref, acc_ref):
    i = pl.program_id(0)
    nsteps = pl.num_programs(0)
    molq = psum_ref[:, 1]
    nat = psum_ref[:, 2]
    meanq = molq / jnp.maximum(nat, 1.0)
    lane = jax.lax.broadcasted_iota(jnp.int32, (_BP, 8), 1)
    meanq_mat = jnp.where(lane == 0, meanq[:, None], 0.0)
    mol = jax.lax.broadcasted_iota(jnp.int32, (_BN, _BP), 1)
    oh = (bidx_ref[:, 0][:, None] == mol).astype(jnp.float32)
    mq = jnp.dot(oh, meanq_mat, preferred_element_type=jnp.float32)[:, 0]
    q_i = qraw_ref[:, 0] - mq
    qi_ref[...] = q_i[:, None]
    vals = q_i[:, None] * posp_ref[...]

    @pl.when(i == 0)
    def _():
        acc_ref[...] = jnp.zeros_like(acc_ref)

    acc_ref[...] += jax.lax.dot_general(
        oh, vals, (((0,), (0,)), ((), ())),
        preferred_element_type=jnp.float32)

    @pl.when(i == nsteps - 1)
    def _():
        mu = acc_ref[...]
        dip_ref[...] = jnp.sqrt(jnp.sum(mu * mu, axis=1) + 1e-12)[:, None]


def _ro2(qraw, bidx2, posp, psum):
    return pl.pallas_call(
        _ro2_body,
        grid=(_N // _BN,),
        in_specs=[
            pl.BlockSpec((_BN, 1), lambda i: (i, 0)),
            pl.BlockSpec((_BN, 1), lambda i: (i, 0)),
            pl.BlockSpec((_BN, 8), lambda i: (i, 0)),
            pl.BlockSpec((_BP, 128), lambda i: (0, 0)),
        ],
        out_specs=[
            pl.BlockSpec((_BN, 1), lambda i: (i, 0)),
            pl.BlockSpec((_BP, 1), lambda i: (0, 0)),
        ],
        out_shape=[
            jax.ShapeDtypeStruct((_N, 1), jnp.float32),
            jax.ShapeDtypeStruct((_BP, 1), jnp.float32),
        ],
        scratch_shapes=[pltpu.VMEM((_BP, 8), jnp.float32)],
    )(qraw, bidx2, posp, psum)


def kernel(z, pos, edge_index, batch_idx, params):
    src = edge_index[0].astype(jnp.int32)
    dst = edge_index[1].astype(jnp.int32)
    npad = _EPAD - _E
    src2 = jnp.concatenate(
        [src, jnp.full((npad,), _NPAD - 1, jnp.int32)]).reshape(_NCHUNK, _CH)
    dst2 = jnp.concatenate(
        [dst, jnp.zeros((npad,), jnp.int32)]).reshape(_NCHUNK, _CH)
    px = jnp.asarray(pos[:, 0], jnp.float32)
    py = jnp.asarray(pos[:, 1], jnp.float32)
    pz = jnp.asarray(pos[:, 2], jnp.float32)
    zeros = jnp.zeros((_RPS, _D), jnp.float32)

    r2 = jnp.concatenate(
        [_sc_r2(px, py, pz, src, dst), jnp.zeros((_EPAD - _E,), jnp.float32)])

    embp = jnp.zeros((128, _D), jnp.float32).at[: _MAXZ + 1].set(params["emb"])
    x = _embed(z.astype(jnp.int32).reshape(_N, 1), embp)

    r2c = r2.reshape(_EPAD, 1)
    for blk in params["blocks"]:
        xd = _sc_gather(x, dst2)
        m = _wm(r2c, xd,
                blk["fw1"], blk["fb1"].reshape(1, _D),
                blk["fw2"], blk["fb2"].reshape(1, _D))
        p0, p1 = _sc_scatter(m, src2, zeros)
        x = _upd(x, p0, p1,
                 blk["uw1"], blk["ub1"].reshape(1, _D),
                 blk["uw2"], blk["ub2"].reshape(1, _D))

    w1 = jnp.concatenate([params["ew1"], params["cw1"]], axis=1)
    b1 = jnp.concatenate([params["eb1"], params["cb1"]], axis=0).reshape(1, 128)
    w2 = jnp.zeros((_D, 128), jnp.float32)
    w2 = w2.at[: _D // 2, 0].set(params["ew2"][:, 0])
    w2 = w2.at[_D // 2 :, 1].set(params["cw2"][:, 0])
    b2 = jnp.zeros((1, 128), jnp.float32)
    b2 = b2.at[0, 0].set(params["eb2"][0])
    b2 = b2.at[0, 1].set(params["cb2"][0])

    bidx2 = batch_idx.astype(jnp.int32).reshape(_N, 1)
    qraw, psum = _ro1(x, bidx2, w1, b1, w2, b2)

    posp = jnp.zeros((_N, 8), jnp.float32).at[:, :3].set(pos)
    qi, dip = _ro2(qraw, bidx2, posp, psum)

    energy = psum[: _B, 0]
    dipole = dip[: _B, 0]
    q_i = qi[:, 0]
    return energy, dipole, q_i


# speedup vs baseline: 2.6986x; 1.3698x over previous
import functools

import jax
import jax.numpy as jnp
import numpy as np
from jax import lax
from jax.experimental import pallas as pl
from jax.experimental.pallas import tpu as pltpu
from jax.experimental.pallas import tpu_sc as plsc

_N = 10000
_E = 320000
_B = 500
_D = 128
_NRBF = 64
_RCUT = 5.0
_MAXZ = 100

_LOG2 = float(np.log(2.0))
_BP = 512
_BN = 2000
_BE = 4096

_OFFS = np.linspace(0.0, _RCUT, _NRBF).astype(np.float32)
_WIDTH = float(_OFFS[1] - _OFFS[0])
_COEF = -0.5 / (_WIDTH * _WIDTH)


def _ssp(x):
    return jnp.maximum(x, 0.0) + jnp.log1p(jnp.exp(-jnp.abs(x))) - _LOG2


_NC = 2
_NS = 16
_NW = _NC * _NS
_CH = 128
_CPT = 80
_NCHUNK = _NW * _CPT
_EPAD = _NCHUNK * _CH
_EPT = _E // _NW
_NPAD = 10240
_RPS = _NPAD // _NS

_sc_mesh = plsc.VectorSubcoreMesh(core_axis_name="c", subcore_axis_name="s")
_sc_params = pltpu.CompilerParams(needs_layout_passes=False)


def _wid():
    return lax.axis_index("s") * _NC + lax.axis_index("c")


def _r2_body(px_hbm, py_hbm, pz_hbm, src_hbm, dst_hbm, out_hbm,
             px, py, pz, srcv, dstv, outv):
    w = _wid()
    base = w * _EPT
    pltpu.sync_copy(px_hbm, px)
    pltpu.sync_copy(py_hbm, py)
    pltpu.sync_copy(pz_hbm, pz)
    pltpu.sync_copy(src_hbm.at[pl.ds(base, _EPT)], srcv)
    pltpu.sync_copy(dst_hbm.at[pl.ds(base, _EPT)], dstv)

    def body(j, carry):
        sl = pl.ds(j * 16, 16)
        si = srcv[sl]
        di = dstv[sl]
        dx = plsc.load_gather(px, [di]) - plsc.load_gather(px, [si])
        dy = plsc.load_gather(py, [di]) - plsc.load_gather(py, [si])
        dz = plsc.load_gather(pz, [di]) - plsc.load_gather(pz, [si])
        outv[sl] = dx * dx + dy * dy + dz * dz
        return carry

    lax.fori_loop(0, _EPT // 16, body, 0)
    pltpu.sync_copy(outv, out_hbm.at[pl.ds(base, _EPT)])


def _sc_r2(px, py, pz, src, dst):
    f = functools.partial(
        pl.kernel,
        out_type=jax.ShapeDtypeStruct((_E,), jnp.float32),
        mesh=_sc_mesh,
        compiler_params=_sc_params,
        scratch_types=[
            pltpu.VMEM((_N,), jnp.float32),
            pltpu.VMEM((_N,), jnp.float32),
            pltpu.VMEM((_N,), jnp.float32),
            pltpu.VMEM((_EPT,), jnp.int32),
            pltpu.VMEM((_EPT,), jnp.int32),
            pltpu.VMEM((_EPT,), jnp.float32),
        ],
    )(_r2_body)
    return f(px, py, pz, src, dst)


_NBUF = 2


def _gather_body(x_hbm, dst2_hbm, out_hbm, xs, idx_v, *bufs_sems):
    rbs = bufs_sems[:_NBUF]
    sgs = bufs_sems[_NBUF:2 * _NBUF]
    sos = bufs_sems[2 * _NBUF:3 * _NBUF]
    w = _wid()
    s = lax.axis_index("s")
    base = w * _CPT

    @pl.when(s < _NS - 1)
    def _():
        pltpu.sync_copy(x_hbm.at[pl.ds(s * _RPS, _RPS)],
                        xs.at[pl.ds(s * _RPS, _RPS)])

    @pl.when(s == _NS - 1)
    def _():
        pltpu.sync_copy(x_hbm.at[pl.ds((_NS - 1) * _RPS, _N - (_NS - 1) * _RPS)],
                        xs.at[pl.ds((_NS - 1) * _RPS, _N - (_NS - 1) * _RPS)])

    pltpu.sync_copy(dst2_hbm.at[pl.ds(base, _CPT)], idx_v)
    plsc.subcore_barrier()

    def out_sl(g):
        return out_hbm.at[pl.ds((base + g) * _CH, _CH)]

    for b in range(_NBUF):
        pltpu.async_copy(xs.at[idx_v.at[b]], rbs[b], sgs[b])

    def body(k, carry):
        for b in range(_NBUF):
            g = k * _NBUF + b
            pltpu.make_async_copy(xs.at[idx_v.at[g]], rbs[b], sgs[b]).wait()
            pltpu.async_copy(rbs[b], out_sl(g), sos[b])

            @pl.when(g + _NBUF < _CPT)
            def _():
                pltpu.make_async_copy(rbs[b], out_sl(g), sos[b]).wait()
                pltpu.async_copy(
                    xs.at[idx_v.at[g + _NBUF]], rbs[b], sgs[b])
        return carry

    lax.fori_loop(0, _CPT // _NBUF, body, 0)
    for b in range(_NBUF):
        pltpu.make_async_copy(rbs[b], out_sl(_CPT - _NBUF + b), sos[b]).wait()


def _sc_gather(x, dst2):
    f = functools.partial(
        pl.kernel,
        out_type=jax.ShapeDtypeStruct((_EPAD, _D), jnp.float32),
        mesh=_sc_mesh,
        compiler_params=_sc_params,
        scratch_types=[
            pltpu.VMEM_SHARED((_NPAD, _D), jnp.float32),
            pltpu.VMEM((_CPT, _CH), jnp.int32),
        ] + [pltpu.VMEM((_CH, _D), jnp.float32)] * _NBUF
          + [pltpu.SemaphoreType.DMA] * (2 * _NBUF),
    )(_gather_body)
    return f(x, dst2)


_NBUF_S = 2


def _scatter_body(m_hbm, src2_hbm, zeros_hbm, p0_hbm, p1_hbm,
                  acc, idx_v, *bufs_sems):
    mbs = bufs_sems[:_NBUF_S]
    sis = bufs_sems[_NBUF_S:2 * _NBUF_S]
    c = lax.axis_index("c")
    s = lax.axis_index("s")
    w = s * _NC + c
    base = w * _CPT
    rsl = pl.ds(s * _RPS, _RPS)
    pltpu.sync_copy(zeros_hbm, acc.at[rsl])
    plsc.subcore_barrier()

    pltpu.sync_copy(src2_hbm.at[pl.ds(base, _CPT)], idx_v)

    def m_sl(g):
        return m_hbm.at[pl.ds((base + g) * _CH, _CH)]

    for b in range(_NBUF_S):
        pltpu.async_copy(m_sl(b), mbs[b], sis[b])

    def body(k, carry):
        for b in range(_NBUF_S):
            g = k * _NBUF_S + b
            pltpu.make_async_copy(m_sl(g), mbs[b], sis[b]).wait()
            pltpu.sync_copy(mbs[b], acc.at[idx_v.at[g]], add=True)

            @pl.when(g + _NBUF_S < _CPT)
            def _():
                pltpu.async_copy(m_sl(g + _NBUF_S), mbs[b], sis[b])
        return carry

    lax.fori_loop(0, _CPT // _NBUF_S, body, 0)

    plsc.subcore_barrier()

    @pl.when(c == 0)
    def _():
        pltpu.sync_copy(acc.at[rsl], p0_hbm.at[rsl])

    @pl.when(c == 1)
    def _():
        pltpu.sync_copy(acc.at[rsl], p1_hbm.at[rsl])


def _sc_scatter(m, src2, zeros):
    f = functools.partial(
        pl.kernel,
        out_type=(
            jax.ShapeDtypeStruct((_NPAD, _D), jnp.float32),
            jax.ShapeDtypeStruct((_NPAD, _D), jnp.float32),
        ),
        mesh=_sc_mesh,
        compiler_params=_sc_params,
        scratch_types=[
            pltpu.VMEM_SHARED((_NPAD, _D), jnp.float32),
            pltpu.VMEM((_CPT, _CH), jnp.int32),
        ] + [pltpu.VMEM((_CH, _D), jnp.float32)] * _NBUF_S
          + [pltpu.SemaphoreType.DMA] * _NBUF_S,
    )(_scatter_body)
    return f(m, src2, zeros)


def _embed_body(z_ref, emb_ref, o_ref):
    zc = z_ref[:, 0][:, None]
    lane = jax.lax.broadcasted_iota(jnp.int32, (_BN, 128), 1)
    oh = (zc == lane).astype(jnp.float32)
    o_ref[...] = jnp.dot(oh, emb_ref[...], preferred_element_type=jnp.float32)


def _embed(z2, embp):
    return pl.pallas_call(
        _embed_body,
        grid=(_N // _BN,),
        in_specs=[
            pl.BlockSpec((_BN, 1), lambda i: (i, 0)),
            pl.BlockSpec((128, _D), lambda i: (0, 0)),
        ],
        out_specs=pl.BlockSpec((_BN, _D), lambda i: (i, 0)),
        out_shape=jax.ShapeDtypeStruct((_N, _D), jnp.float32),
    )(z2, embp)


def _wm_body(r2_ref, xd_ref, fw1_ref, fb1_ref, fw2_ref, fb2_ref, o_ref):
    r = jnp.sqrt(r2_ref[:, 0] + 1e-12)
    offs = (jax.lax.broadcasted_iota(jnp.int32, (_BE, _NRBF), 1)
            .astype(jnp.float32) * _WIDTH)
    e = jnp.exp(_COEF * (r[:, None] - offs) ** 2)
    fc = 0.5 * (jnp.cos((np.pi / _RCUT) * r) + 1.0) * (r < _RCUT).astype(jnp.float32)
    e = e * fc[:, None]
    a = _ssp(jnp.dot(e, fw1_ref[...], preferred_element_type=jnp.float32)
             + fb1_ref[...])
    w = jnp.dot(a, fw2_ref[...], preferred_element_type=jnp.float32) + fb2_ref[...]
    o_ref[...] = w * xd_ref[...]


def _wm(r2c, xd, fw1, fb1, fw2, fb2):
    return pl.pallas_call(
        _wm_body,
        grid=(_EPAD // _BE,),
        in_specs=[
            pl.BlockSpec((_BE, 1), lambda i: (i, 0)),
            pl.BlockSpec((_BE, _D), lambda i: (i, 0)),
            pl.BlockSpec((_NRBF, _D), lambda i: (0, 0)),
            pl.BlockSpec((1, _D), lambda i: (0, 0)),
            pl.BlockSpec((_D, _D), lambda i: (0, 0)),
            pl.BlockSpec((1, _D), lambda i: (0, 0)),
        ],
        out_specs=pl.BlockSpec((_BE, _D), lambda i: (i, 0)),
        out_shape=jax.ShapeDtypeStruct((_EPAD, _D), jnp.float32),
    )(r2c, xd, fw1, fb1, fw2, fb2)


def _upd_body(x_ref, p0_ref, p1_ref, uw1_ref, ub1_ref, uw2_ref, ub2_ref, o_ref):
    mi = p0_ref[...] + p1_ref[...]
    t = _ssp(jnp.dot(mi, uw1_ref[...], preferred_element_type=jnp.float32)
             + ub1_ref[...])
    h = jnp.dot(t, uw2_ref[...], preferred_element_type=jnp.float32) + ub2_ref[...]
    o_ref[...] = x_ref[...] + h


def _upd(x, p0, p1, uw1, ub1, uw2, ub2):
    return pl.pallas_call(
        _upd_body,
        grid=(_N // _BN,),
        in_specs=[
            pl.BlockSpec((_BN, _D), lambda i: (i, 0)),
            pl.BlockSpec((_BN, _D), lambda i: (i, 0)),
            pl.BlockSpec((_BN, _D), lambda i: (i, 0)),
            pl.BlockSpec((_D, _D), lambda i: (0, 0)),
            pl.BlockSpec((1, _D), lambda i: (0, 0)),
            pl.BlockSpec((_D, _D), lambda i: (0, 0)),
            pl.BlockSpec((1, _D), lambda i: (0, 0)),
        ],
        out_specs=pl.BlockSpec((_BN, _D), lambda i: (i, 0)),
        out_shape=jax.ShapeDtypeStruct((_N, _D), jnp.float32),
    )(x, p0, p1, uw1, ub1, uw2, ub2)


def _ro1_body(x_ref, bidx_ref, w1_ref, b1_ref, w2_ref, b2_ref,
              qraw_ref, psum_ref):
    i = pl.program_id(0)
    t = _ssp(jnp.dot(x_ref[...], w1_ref[...], preferred_element_type=jnp.float32)
             + b1_ref[...])
    u = jnp.dot(t, w2_ref[...], preferred_element_type=jnp.float32) + b2_ref[...]
    eps = u[:, 0]
    q = u[:, 1]
    qraw_ref[...] = q[:, None]
    lane = jax.lax.broadcasted_iota(jnp.int32, (_BN, 128), 1)
    vals = jnp.where(lane == 0, eps[:, None],
                     jnp.where(lane == 1, q[:, None],
                               jnp.where(lane == 2, 1.0, 0.0)))
    mol = jax.lax.broadcasted_iota(jnp.int32, (_BN, _BP), 1)
    oh = (bidx_ref[:, 0][:, None] == mol).astype(jnp.float32)

    @pl.when(i == 0)
    def _():
        psum_ref[...] = jnp.zeros_like(psum_ref)

    psum_ref[...] += jax.lax.dot_general(
        oh, vals, (((0,), (0,)), ((), ())),
        preferred_element_type=jnp.float32)


def _ro1(x, bidx2, w1, b1, w2, b2):
    return pl.pallas_call(
        _ro1_body,
        grid=(_N // _BN,),
        in_specs=[
            pl.BlockSpec((_BN, _D), lambda i: (i, 0)),
            pl.BlockSpec((_BN, 1), lambda i: (i, 0)),
            pl.BlockSpec((_D, _D), lambda i: (0, 0)),
            pl.BlockSpec((1, _D), lambda i: (0, 0)),
            pl.BlockSpec((_D, 128), lambda i: (0, 0)),
            pl.BlockSpec((1, 128), lambda i: (0, 0)),
        ],
        out_specs=[
            pl.BlockSpec((_BN, 1), lambda i: (i, 0)),
            pl.BlockSpec((_BP, 128), lambda i: (0, 0)),
        ],
        out_shape=[
            jax.ShapeDtypeStruct((_N, 1), jnp.float32),
            jax.ShapeDtypeStruct((_BP, 128), jnp.float32),
        ],
    )(x, bidx2, w1, b1, w2, b2)


def _ro2_body(qraw_ref, bidx_ref, posp_ref, psum_ref,
              qi_ref, dip_ref, acc_ref):
    i = pl.program_id(0)
    nsteps = pl.num_programs(0)
    molq = psum_ref[:, 1]
    nat = psum_ref[:, 2]
    meanq = molq / jnp.maximum(nat, 1.0)
    lane = jax.lax.broadcasted_iota(jnp.int32, (_BP, 8), 1)
    meanq_mat = jnp.where(lane == 0, meanq[:, None], 0.0)
    mol = jax.lax.broadcasted_iota(jnp.int32, (_BN, _BP), 1)
    oh = (bidx_ref[:, 0][:, None] == mol).astype(jnp.float32)
    mq = jnp.dot(oh, meanq_mat, preferred_element_type=jnp.float32)[:, 0]
    q_i = qraw_ref[:, 0] - mq
    qi_ref[...] = q_i[:, None]
    vals = q_i[:, None] * posp_ref[...]

    @pl.when(i == 0)
    def _():
        acc_ref[...] = jnp.zeros_like(acc_ref)

    acc_ref[...] += jax.lax.dot_general(
        oh, vals, (((0,), (0,)), ((), ())),
        preferred_element_type=jnp.float32)

    @pl.when(i == nsteps - 1)
    def _():
        mu = acc_ref[...]
        dip_ref[...] = jnp.sqrt(jnp.sum(mu * mu, axis=1) + 1e-12)[:, None]


def _ro2(qraw, bidx2, posp, psum):
    return pl.pallas_call(
        _ro2_body,
        grid=(_N // _BN,),
        in_specs=[
            pl.BlockSpec((_BN, 1), lambda i: (i, 0)),
            pl.BlockSpec((_BN, 1), lambda i: (i, 0)),
            pl.BlockSpec((_BN, 8), lambda i: (i, 0)),
            pl.BlockSpec((_BP, 128), lambda i: (0, 0)),
        ],
        out_specs=[
            pl.BlockSpec((_BN, 1), lambda i: (i, 0)),
            pl.BlockSpec((_BP, 1), lambda i: (0, 0)),
        ],
        out_shape=[
            jax.ShapeDtypeStruct((_N, 1), jnp.float32),
            jax.ShapeDtypeStruct((_BP, 1), jnp.float32),
        ],
        scratch_shapes=[pltpu.VMEM((_BP, 8), jnp.float32)],
    )(qraw, bidx2, posp, psum)


def kernel(z, pos, edge_index, batch_idx, params):
    src = edge_index[0].astype(jnp.int32)
    dst = edge_index[1].astype(jnp.int32)
    npad = _EPAD - _E
    src2 = jnp.concatenate(
        [src, jnp.full((npad,), _NPAD - 1, jnp.int32)]).reshape(_NCHUNK, _CH)
    dst2 = jnp.concatenate(
        [dst, jnp.zeros((npad,), jnp.int32)]).reshape(_NCHUNK, _CH)
    px = jnp.asarray(pos[:, 0], jnp.float32)
    py = jnp.asarray(pos[:, 1], jnp.float32)
    pz = jnp.asarray(pos[:, 2], jnp.float32)
    zeros = jnp.zeros((_RPS, _D), jnp.float32)

    r2 = jnp.concatenate(
        [_sc_r2(px, py, pz, src, dst), jnp.zeros((_EPAD - _E,), jnp.float32)])

    embp = jnp.zeros((128, _D), jnp.float32).at[: _MAXZ + 1].set(params["emb"])
    x = _embed(z.astype(jnp.int32).reshape(_N, 1), embp)

    r2c = r2.reshape(_EPAD, 1)
    for blk in params["blocks"]:
        xd = _sc_gather(x, dst2)
        m = _wm(r2c, xd,
                blk["fw1"], blk["fb1"].reshape(1, _D),
                blk["fw2"], blk["fb2"].reshape(1, _D))
        p0, p1 = _sc_scatter(m, src2, zeros)
        x = _upd(x, p0, p1,
                 blk["uw1"], blk["ub1"].reshape(1, _D),
                 blk["uw2"], blk["ub2"].reshape(1, _D))

    w1 = jnp.concatenate([params["ew1"], params["cw1"]], axis=1)
    b1 = jnp.concatenate([params["eb1"], params["cb1"]], axis=0).reshape(1, 128)
    w2 = jnp.zeros((_D, 128), jnp.float32)
    w2 = w2.at[: _D // 2, 0].set(params["ew2"][:, 0])
    w2 = w2.at[_D // 2 :, 1].set(params["cw2"][:, 0])
    b2 = jnp.zeros((1, 128), jnp.float32)
    b2 = b2.at[0, 0].set(params["eb2"][0])
    b2 = b2.at[0, 1].set(params["cb2"][0])

    bidx2 = batch_idx.astype(jnp.int32).reshape(_N, 1)
    qraw, psum = _ro1(x, bidx2, w1, b1, w2, b2)

    posp = jnp.zeros((_N, 8), jnp.float32).at[:, :3].set(pos)
    qi, dip = _ro2(qraw, bidx2, posp, psum)

    energy = psum[: _B, 0]
    dipole = dip[: _B, 0]
    q_i = qi[:, 0]
    return energy, dipole, q_i
